# baseline (speedup 1.0000x reference)
import functools

import jax
import jax.numpy as jnp
from jax import lax
from jax.experimental import pallas as pl
from jax.experimental.pallas import tpu as pltpu
from jax.experimental.pallas import tpu_sc as plsc

N = 10000
E = 160000
D = 256
H = 512

NC = 2
NS = 16
NW = NC * NS
EPW = 5120
K = 128
NB = EPW // K
NPAD = 10240
RPT = NPAD // NS

_mesh = plsc.VectorSubcoreMesh(
    core_axis_name="c", subcore_axis_name="s", num_cores=NC, num_subcores=NS)

f32 = jnp.float32


def _deg_kernel_fn():
  @functools.partial(
      pl.kernel,
      out_type=jax.ShapeDtypeStruct((NC, NPAD, 128), f32),
      mesh=_mesh,
      scratch_types=[
          pltpu.VMEM((NB, K), jnp.int32),
          pltpu.VMEM((K, 128), f32),
          pltpu.VMEM_SHARED((NPAD, 128), f32),
      ],
  )
  def deg_kernel(dst_hbm, zeros_hbm, ones_hbm, out_hbm, dst_v, ones_v, acc):
    c = lax.axis_index("c")
    s = lax.axis_index("s")
    pltpu.sync_copy(dst_hbm.at[c, s], dst_v)
    pltpu.sync_copy(ones_hbm, ones_v)
    pltpu.sync_copy(zeros_hbm, acc.at[pl.ds(s * RPT, RPT)])
    plsc.subcore_barrier()

    def body(j, carry):
      pltpu.sync_copy(ones_v, acc.at[dst_v.at[j]], add=True)
      return carry

    lax.fori_loop(0, NB, body, 0)
    plsc.subcore_barrier()
    pltpu.sync_copy(acc.at[pl.ds(s * RPT, RPT)],
                    out_hbm.at[c, pl.ds(s * RPT, RPT)])

  return deg_kernel


def _scatter_kernel_fn(num_chunks):
  @functools.partial(
      pl.kernel,
      out_type=jax.ShapeDtypeStruct((NC, num_chunks, NPAD, 128), f32),
      mesh=_mesh,
      scratch_types=[
          pltpu.VMEM((NB, K), jnp.int32),
          pltpu.VMEM((NB, K), jnp.int32),
          pltpu.VMEM((K, 128), f32),
          pltpu.VMEM((K, 128), f32),
          pltpu.VMEM_SHARED((NPAD, 128), f32),
          pltpu.SemaphoreType.DMA,
          pltpu.SemaphoreType.DMA,
      ],
  )
  def scatter_kernel(m_hbm, src_hbm, dst_hbm, zeros_hbm, out_hbm,
                     src_v, dst_v, rows_a, rows_b, acc, sem_a, sem_b):
    c = lax.axis_index("c")
    s = lax.axis_index("s")
    pltpu.sync_copy(src_hbm.at[c, s], src_v)
    pltpu.sync_copy(dst_hbm.at[c, s], dst_v)

    for cc in range(num_chunks):
      m_view = m_hbm.at[cc]
      pltpu.sync_copy(zeros_hbm, acc.at[pl.ds(s * RPT, RPT)])
      plsc.subcore_barrier()

      def start_g(j, buf, sem):
        pltpu.async_copy(m_view.at[src_v.at[j]], buf, sem)

      def finish_g(j, buf, sem):
        pltpu.make_async_copy(m_view.at[src_v.at[j]], buf, sem).wait()

      def step(j, buf, sem):
        finish_g(j, buf, sem)
        pltpu.sync_copy(buf, acc.at[dst_v.at[j]], add=True)

      start_g(0, rows_a, sem_a)

      def body(j, carry):
        @pl.when(j % 2 == 0)
        def _():
          @pl.when(j + 1 < NB)
          def _():
            start_g(j + 1, rows_b, sem_b)
          step(j, rows_a, sem_a)

        @pl.when(j % 2 == 1)
        def _():
          @pl.when(j + 1 < NB)
          def _():
            start_g(j + 1, rows_a, sem_a)
          step(j, rows_b, sem_b)

        return carry

      lax.fori_loop(0, NB, body, 0)
      plsc.subcore_barrier()
      pltpu.sync_copy(acc.at[pl.ds(s * RPT, RPT)],
                      out_hbm.at[c, cc, pl.ds(s * RPT, RPT)])
      plsc.subcore_barrier()

  return scatter_kernel


_deg_call = _deg_kernel_fn()
_scatter4 = _scatter_kernel_fn(4)
_scatter2 = _scatter_kernel_fn(2)

RB = 1000


def _mm1_body(x_ref, w_ref, dinv_ref, o_ref):
  xs = x_ref[...] * dinv_ref[...]
  o_ref[0] = jnp.dot(xs, w_ref[...], preferred_element_type=f32)


def _mm2_body(hp_ref, m1_ref, dinv_ref, b1_ref, w2_ref, o_ref):
  k = pl.program_id(2)
  h = ((hp_ref[0, 0] + hp_ref[1, 0] + m1_ref[0]) * dinv_ref[...]
       + b1_ref[...][None, :])
  part = jnp.dot(h, w2_ref[...], preferred_element_type=f32)

  @pl.when(k == 0)
  def _():
    o_ref[0] = part

  @pl.when(k > 0)
  def _():
    o_ref[0] += part

  @pl.when(k == (H // 128) - 1)
  def _():
    o_ref[0] *= dinv_ref[...]


def _fin_body(gp_ref, m2_ref, dinv_ref, b2_ref, o_ref):
  o_ref[...] = ((gp_ref[0, 0] + gp_ref[1, 0] + m2_ref[0])
                * dinv_ref[...] + b2_ref[...][None, :])


def kernel(embedding, W1, b1, W2, b2, edge_index):
  pad = NW * EPW - E
  src_p = jnp.concatenate(
      [edge_index[0], jnp.zeros((pad,), jnp.int32)])
  dst_p = jnp.concatenate(
      [edge_index[1],
       N + (jnp.arange(pad, dtype=jnp.int32) % (NPAD - N))])
  src = src_p.reshape(NC, NS, NB, K)
  dst = dst_p.reshape(NC, NS, NB, K)

  zeros_acc = jnp.zeros((RPT, 128), f32)
  ones_deg = jnp.ones((K, 128), f32)

  degp = _deg_call(dst, zeros_acc, ones_deg)
  deg = 1.0 + degp[0, :, 0] + degp[1, :, 0]
  dinv = lax.rsqrt(deg)
  dinv_c = dinv.reshape(NPAD, 1)

  m1 = pl.pallas_call(
      _mm1_body,
      grid=(N // RB, H // 128),
      in_specs=[
          pl.BlockSpec((RB, D), lambda i, c: (i, 0)),
          pl.BlockSpec((D, 128), lambda i, c: (0, c)),
          pl.BlockSpec((RB, 1), lambda i, c: (i, 0)),
      ],
      out_specs=pl.BlockSpec((1, RB, 128), lambda i, c: (c, i, 0)),
      out_shape=jax.ShapeDtypeStruct((H // 128, N, 128), f32),
  )(embedding, W1, dinv_c)

  acc1 = _scatter4(m1, src, dst, zeros_acc)

  m2 = pl.pallas_call(
      _mm2_body,
      grid=(N // RB, D // 128, H // 128),
      in_specs=[
          pl.BlockSpec((NC, 1, RB, 128), lambda i, c, k: (0, k, i, 0)),
          pl.BlockSpec((1, RB, 128), lambda i, c, k: (k, i, 0)),
          pl.BlockSpec((RB, 1), lambda i, c, k: (i, 0)),
          pl.BlockSpec((128,), lambda i, c, k: (k,)),
          pl.BlockSpec((128, 128), lambda i, c, k: (k, c)),
      ],
      out_specs=pl.BlockSpec((1, RB, 128), lambda i, c, k: (c, i, 0)),
      out_shape=jax.ShapeDtypeStruct((D // 128, N, 128), f32),
  )(acc1, m1, dinv_c, b1, W2)

  acc2 = _scatter2(m2, src, dst, zeros_acc)

  out = pl.pallas_call(
      _fin_body,
      grid=(N // RB, D // 128),
      in_specs=[
          pl.BlockSpec((NC, 1, RB, 128), lambda i, c: (0, c, i, 0)),
          pl.BlockSpec((1, RB, 128), lambda i, c: (c, i, 0)),
          pl.BlockSpec((RB, 1), lambda i, c: (i, 0)),
          pl.BlockSpec((128,), lambda i, c: (c,)),
      ],
      out_specs=pl.BlockSpec((RB, 128), lambda i, c: (i, c)),
      out_shape=jax.ShapeDtypeStruct((N, D), f32),
  )(acc2, m2, dinv_c, b2)

  return out


# speedup vs baseline: 1.0577x; 1.0577x over previous
import functools

import jax
import jax.numpy as jnp
from jax import lax
from jax.experimental import pallas as pl
from jax.experimental.pallas import tpu as pltpu
from jax.experimental.pallas import tpu_sc as plsc

N = 10000
E = 160000
D = 256
H = 512

NC = 2
NS = 16
NW = NC * NS
EPW = 5120
K = 64
NB = EPW // K
NPAD = 10240
RPT = NPAD // NS

_mesh = plsc.VectorSubcoreMesh(
    core_axis_name="c", subcore_axis_name="s", num_cores=NC, num_subcores=NS)

f32 = jnp.float32


def _deg_kernel_fn():
  @functools.partial(
      pl.kernel,
      out_type=jax.ShapeDtypeStruct((NC, NPAD, 128), f32),
      mesh=_mesh,
      scratch_types=[
          pltpu.VMEM((NB, K), jnp.int32),
          pltpu.VMEM((K, 128), f32),
          pltpu.VMEM_SHARED((NPAD, 128), f32),
      ],
  )
  def deg_kernel(dst_hbm, zeros_hbm, ones_hbm, out_hbm, dst_v, ones_v, acc):
    c = lax.axis_index("c")
    s = lax.axis_index("s")
    pltpu.sync_copy(dst_hbm.at[c, s], dst_v)
    pltpu.sync_copy(ones_hbm, ones_v)
    pltpu.sync_copy(zeros_hbm, acc.at[pl.ds(s * RPT, RPT)])
    plsc.subcore_barrier()

    def body(j, carry):
      pltpu.sync_copy(ones_v, acc.at[dst_v.at[j]], add=True)
      return carry

    lax.fori_loop(0, NB, body, 0)
    plsc.subcore_barrier()
    pltpu.sync_copy(acc.at[pl.ds(s * RPT, RPT)],
                    out_hbm.at[c, pl.ds(s * RPT, RPT)])

  return deg_kernel


def _scatter_kernel_fn(num_chunks):
  @functools.partial(
      pl.kernel,
      out_type=jax.ShapeDtypeStruct((NC, num_chunks, NPAD, 128), f32),
      mesh=_mesh,
      scratch_types=[
          pltpu.VMEM((NB, K), jnp.int32),
          pltpu.VMEM((NB, K), jnp.int32),
          pltpu.VMEM((K, 128), f32),
          pltpu.VMEM((K, 128), f32),
          pltpu.VMEM_SHARED((NPAD, 128), f32),
          pltpu.SemaphoreType.DMA,
          pltpu.SemaphoreType.DMA,
      ],
  )
  def scatter_kernel(m_hbm, src_hbm, dst_hbm, zeros_hbm, out_hbm,
                     src_v, dst_v, rows_a, rows_b, acc, sem_a, sem_b):
    c = lax.axis_index("c")
    s = lax.axis_index("s")
    pltpu.sync_copy(src_hbm.at[c, s], src_v)
    pltpu.sync_copy(dst_hbm.at[c, s], dst_v)

    for cc in range(num_chunks):
      m_view = m_hbm.at[cc]
      pltpu.sync_copy(zeros_hbm, acc.at[pl.ds(s * RPT, RPT)])
      plsc.subcore_barrier()

      def start_g(j, buf, sem):
        pltpu.async_copy(m_view.at[src_v.at[j]], buf, sem)

      def finish_g(j, buf, sem):
        pltpu.make_async_copy(m_view.at[src_v.at[j]], buf, sem).wait()

      def step(j, buf, sem):
        finish_g(j, buf, sem)
        pltpu.sync_copy(buf, acc.at[dst_v.at[j]], add=True)

      start_g(0, rows_a, sem_a)

      def body(j, carry):
        @pl.when(j % 2 == 0)
        def _():
          @pl.when(j + 1 < NB)
          def _():
            start_g(j + 1, rows_b, sem_b)
          step(j, rows_a, sem_a)

        @pl.when(j % 2 == 1)
        def _():
          @pl.when(j + 1 < NB)
          def _():
            start_g(j + 1, rows_a, sem_a)
          step(j, rows_b, sem_b)

        return carry

      lax.fori_loop(0, NB, body, 0)
      plsc.subcore_barrier()
      pltpu.sync_copy(acc.at[pl.ds(s * RPT, RPT)],
                      out_hbm.at[c, cc, pl.ds(s * RPT, RPT)])
      plsc.subcore_barrier()

  return scatter_kernel


_deg_call = _deg_kernel_fn()
_scatter4 = _scatter_kernel_fn(4)
_scatter2 = _scatter_kernel_fn(2)

RB = 1000


def _mm1_body(x_ref, w_ref, dinv_ref, o_ref):
  xs = x_ref[...] * dinv_ref[...]
  o_ref[0] = jnp.dot(xs, w_ref[...], preferred_element_type=f32)


def _mm2_body(hp_ref, m1_ref, dinv_ref, b1_ref, w2_ref, o_ref):
  k = pl.program_id(2)
  h = ((hp_ref[0, 0] + hp_ref[1, 0] + m1_ref[0]) * dinv_ref[...]
       + b1_ref[...][None, :])
  part = jnp.dot(h, w2_ref[...], preferred_element_type=f32)

  @pl.when(k == 0)
  def _():
    o_ref[0] = part

  @pl.when(k > 0)
  def _():
    o_ref[0] += part

  @pl.when(k == (H // 128) - 1)
  def _():
    o_ref[0] *= dinv_ref[...]


def _fin_body(gp_ref, m2_ref, dinv_ref, b2_ref, o_ref):
  o_ref[...] = ((gp_ref[0, 0] + gp_ref[1, 0] + m2_ref[0])
                * dinv_ref[...] + b2_ref[...][None, :])


def kernel(embedding, W1, b1, W2, b2, edge_index):
  pad = NW * EPW - E
  src_p = jnp.concatenate(
      [edge_index[0], jnp.zeros((pad,), jnp.int32)])
  dst_p = jnp.concatenate(
      [edge_index[1],
       N + (jnp.arange(pad, dtype=jnp.int32) % (NPAD - N))])
  src = src_p.reshape(NC, NS, NB, K)
  dst = dst_p.reshape(NC, NS, NB, K)

  zeros_acc = jnp.zeros((RPT, 128), f32)
  ones_deg = jnp.ones((K, 128), f32)

  degp = _deg_call(dst, zeros_acc, ones_deg)
  deg = 1.0 + degp[0, :, 0] + degp[1, :, 0]
  dinv = lax.rsqrt(deg)
  dinv_c = dinv.reshape(NPAD, 1)

  m1 = pl.pallas_call(
      _mm1_body,
      grid=(N // RB, H // 128),
      in_specs=[
          pl.BlockSpec((RB, D), lambda i, c: (i, 0)),
          pl.BlockSpec((D, 128), lambda i, c: (0, c)),
          pl.BlockSpec((RB, 1), lambda i, c: (i, 0)),
      ],
      out_specs=pl.BlockSpec((1, RB, 128), lambda i, c: (c, i, 0)),
      out_shape=jax.ShapeDtypeStruct((H // 128, N, 128), f32),
  )(embedding, W1, dinv_c)

  acc1 = _scatter4(m1, src, dst, zeros_acc)

  m2 = pl.pallas_call(
      _mm2_body,
      grid=(N // RB, D // 128, H // 128),
      in_specs=[
          pl.BlockSpec((NC, 1, RB, 128), lambda i, c, k: (0, k, i, 0)),
          pl.BlockSpec((1, RB, 128), lambda i, c, k: (k, i, 0)),
          pl.BlockSpec((RB, 1), lambda i, c, k: (i, 0)),
          pl.BlockSpec((128,), lambda i, c, k: (k,)),
          pl.BlockSpec((128, 128), lambda i, c, k: (k, c)),
      ],
      out_specs=pl.BlockSpec((1, RB, 128), lambda i, c, k: (c, i, 0)),
      out_shape=jax.ShapeDtypeStruct((D // 128, N, 128), f32),
  )(acc1, m1, dinv_c, b1, W2)

  acc2 = _scatter2(m2, src, dst, zeros_acc)

  out = pl.pallas_call(
      _fin_body,
      grid=(N // RB, D // 128),
      in_specs=[
          pl.BlockSpec((NC, 1, RB, 128), lambda i, c: (0, c, i, 0)),
          pl.BlockSpec((1, RB, 128), lambda i, c: (c, i, 0)),
          pl.BlockSpec((RB, 1), lambda i, c: (i, 0)),
          pl.BlockSpec((128,), lambda i, c: (c,)),
      ],
      out_specs=pl.BlockSpec((RB, 128), lambda i, c: (i, c)),
      out_shape=jax.ShapeDtypeStruct((N, D), f32),
  )(acc2, m2, dinv_c, b2)

  return out


# speedup vs baseline: 1.1167x; 1.0558x over previous
import functools

import jax
import jax.numpy as jnp
from jax import lax
from jax.experimental import pallas as pl
from jax.experimental.pallas import tpu as pltpu
from jax.experimental.pallas import tpu_sc as plsc

N = 10000
E = 160000
D = 256
H = 512

NC = 2
NS = 16
NW = NC * NS
EPW = 5120
K = 40
NB = EPW // K
NPAD = 10240
RPT = NPAD // NS

_mesh = plsc.VectorSubcoreMesh(
    core_axis_name="c", subcore_axis_name="s", num_cores=NC, num_subcores=NS)

f32 = jnp.float32


def _deg_kernel_fn():
  @functools.partial(
      pl.kernel,
      out_type=jax.ShapeDtypeStruct((NC, NPAD, 128), f32),
      mesh=_mesh,
      scratch_types=[
          pltpu.VMEM((NB, K), jnp.int32),
          pltpu.VMEM((K, 128), f32),
          pltpu.VMEM_SHARED((NPAD, 128), f32),
      ],
  )
  def deg_kernel(dst_hbm, zeros_hbm, ones_hbm, out_hbm, dst_v, ones_v, acc):
    c = lax.axis_index("c")
    s = lax.axis_index("s")
    pltpu.sync_copy(dst_hbm.at[c, s], dst_v)
    pltpu.sync_copy(ones_hbm, ones_v)
    pltpu.sync_copy(zeros_hbm, acc.at[pl.ds(s * RPT, RPT)])
    plsc.subcore_barrier()

    def body(j, carry):
      pltpu.sync_copy(ones_v, acc.at[dst_v.at[j]], add=True)
      return carry

    lax.fori_loop(0, NB, body, 0)
    plsc.subcore_barrier()
    pltpu.sync_copy(acc.at[pl.ds(s * RPT, RPT)],
                    out_hbm.at[c, pl.ds(s * RPT, RPT)])

  return deg_kernel


def _scatter_kernel_fn(num_chunks):
  @functools.partial(
      pl.kernel,
      out_type=jax.ShapeDtypeStruct((NC, num_chunks, NPAD, 128), f32),
      mesh=_mesh,
      scratch_types=[
          pltpu.VMEM((NB, K), jnp.int32),
          pltpu.VMEM((NB, K), jnp.int32),
          pltpu.VMEM((K, 128), f32),
          pltpu.VMEM((K, 128), f32),
          pltpu.VMEM_SHARED((NPAD, 128), f32),
          pltpu.SemaphoreType.DMA,
          pltpu.SemaphoreType.DMA,
      ],
  )
  def scatter_kernel(m_hbm, src_hbm, dst_hbm, zeros_hbm, out_hbm,
                     src_v, dst_v, rows_a, rows_b, acc, sem_a, sem_b):
    c = lax.axis_index("c")
    s = lax.axis_index("s")
    pltpu.sync_copy(src_hbm.at[c, s], src_v)
    pltpu.sync_copy(dst_hbm.at[c, s], dst_v)

    for cc in range(num_chunks):
      m_view = m_hbm.at[cc]
      pltpu.sync_copy(zeros_hbm, acc.at[pl.ds(s * RPT, RPT)])
      plsc.subcore_barrier()

      def start_g(j, buf, sem):
        pltpu.async_copy(m_view.at[src_v.at[j]], buf, sem)

      def finish_g(j, buf, sem):
        pltpu.make_async_copy(m_view.at[src_v.at[j]], buf, sem).wait()

      def step(j, buf, sem):
        finish_g(j, buf, sem)
        pltpu.sync_copy(buf, acc.at[dst_v.at[j]], add=True)

      start_g(0, rows_a, sem_a)

      def body(j, carry):
        @pl.when(j % 2 == 0)
        def _():
          @pl.when(j + 1 < NB)
          def _():
            start_g(j + 1, rows_b, sem_b)
          step(j, rows_a, sem_a)

        @pl.when(j % 2 == 1)
        def _():
          @pl.when(j + 1 < NB)
          def _():
            start_g(j + 1, rows_a, sem_a)
          step(j, rows_b, sem_b)

        return carry

      lax.fori_loop(0, NB, body, 0)
      plsc.subcore_barrier()
      pltpu.sync_copy(acc.at[pl.ds(s * RPT, RPT)],
                      out_hbm.at[c, cc, pl.ds(s * RPT, RPT)])
      plsc.subcore_barrier()

  return scatter_kernel


_deg_call = _deg_kernel_fn()
_scatter4 = _scatter_kernel_fn(4)
_scatter2 = _scatter_kernel_fn(2)

RB = 1000


def _mm1_body(x_ref, w_ref, dinv_ref, o_ref):
  xs = x_ref[...] * dinv_ref[...]
  o_ref[0] = jnp.dot(xs, w_ref[...], preferred_element_type=f32)


def _mm2_body(hp_ref, m1_ref, dinv_ref, b1_ref, w2_ref, o_ref):
  k = pl.program_id(2)
  h = ((hp_ref[0, 0] + hp_ref[1, 0] + m1_ref[0]) * dinv_ref[...]
       + b1_ref[...][None, :])
  part = jnp.dot(h, w2_ref[...], preferred_element_type=f32)

  @pl.when(k == 0)
  def _():
    o_ref[0] = part

  @pl.when(k > 0)
  def _():
    o_ref[0] += part

  @pl.when(k == (H // 128) - 1)
  def _():
    o_ref[0] *= dinv_ref[...]


def _fin_body(gp_ref, m2_ref, dinv_ref, b2_ref, o_ref):
  o_ref[...] = ((gp_ref[0, 0] + gp_ref[1, 0] + m2_ref[0])
                * dinv_ref[...] + b2_ref[...][None, :])


def kernel(embedding, W1, b1, W2, b2, edge_index):
  pad = NW * EPW - E
  src_p = jnp.concatenate(
      [edge_index[0], jnp.zeros((pad,), jnp.int32)])
  dst_p = jnp.concatenate(
      [edge_index[1],
       N + (jnp.arange(pad, dtype=jnp.int32) % (NPAD - N))])
  src = src_p.reshape(NC, NS, NB, K)
  dst = dst_p.reshape(NC, NS, NB, K)

  zeros_acc = jnp.zeros((RPT, 128), f32)
  ones_deg = jnp.ones((K, 128), f32)

  degp = _deg_call(dst, zeros_acc, ones_deg)
  deg = 1.0 + degp[0, :, 0] + degp[1, :, 0]
  dinv = lax.rsqrt(deg)
  dinv_c = dinv.reshape(NPAD, 1)

  m1 = pl.pallas_call(
      _mm1_body,
      grid=(N // RB, H // 128),
      in_specs=[
          pl.BlockSpec((RB, D), lambda i, c: (i, 0)),
          pl.BlockSpec((D, 128), lambda i, c: (0, c)),
          pl.BlockSpec((RB, 1), lambda i, c: (i, 0)),
      ],
      out_specs=pl.BlockSpec((1, RB, 128), lambda i, c: (c, i, 0)),
      out_shape=jax.ShapeDtypeStruct((H // 128, N, 128), f32),
  )(embedding, W1, dinv_c)

  acc1 = _scatter4(m1, src, dst, zeros_acc)

  m2 = pl.pallas_call(
      _mm2_body,
      grid=(N // RB, D // 128, H // 128),
      in_specs=[
          pl.BlockSpec((NC, 1, RB, 128), lambda i, c, k: (0, k, i, 0)),
          pl.BlockSpec((1, RB, 128), lambda i, c, k: (k, i, 0)),
          pl.BlockSpec((RB, 1), lambda i, c, k: (i, 0)),
          pl.BlockSpec((128,), lambda i, c, k: (k,)),
          pl.BlockSpec((128, 128), lambda i, c, k: (k, c)),
      ],
      out_specs=pl.BlockSpec((1, RB, 128), lambda i, c, k: (c, i, 0)),
      out_shape=jax.ShapeDtypeStruct((D // 128, N, 128), f32),
  )(acc1, m1, dinv_c, b1, W2)

  acc2 = _scatter2(m2, src, dst, zeros_acc)

  out = pl.pallas_call(
      _fin_body,
      grid=(N // RB, D // 128),
      in_specs=[
          pl.BlockSpec((NC, 1, RB, 128), lambda i, c: (0, c, i, 0)),
          pl.BlockSpec((1, RB, 128), lambda i, c: (c, i, 0)),
          pl.BlockSpec((RB, 1), lambda i, c: (i, 0)),
          pl.BlockSpec((128,), lambda i, c: (c,)),
      ],
      out_specs=pl.BlockSpec((RB, 128), lambda i, c: (i, c)),
      out_shape=jax.ShapeDtypeStruct((N, D), f32),
  )(acc2, m2, dinv_c, b2)

  return out


# speedup vs baseline: 3.1358x; 2.8082x over previous
import functools

import jax
import jax.numpy as jnp
from jax import lax
from jax.experimental import pallas as pl
from jax.experimental.pallas import tpu as pltpu
from jax.experimental.pallas import tpu_sc as plsc

N = 10000
E = 160000
D = 256
H = 512

NC = 2
NS = 16
NW = NC * NS
K = 80
EPT = E // NS
NB = EPT // K
NPAD = 10240
RPT = NPAD // NS

_mesh = plsc.VectorSubcoreMesh(
    core_axis_name="c", subcore_axis_name="s", num_cores=NC, num_subcores=NS)

f32 = jnp.float32


def _deg_kernel_fn():
  @functools.partial(
      pl.kernel,
      out_type=jax.ShapeDtypeStruct((NC, NPAD, 128), f32),
      mesh=_mesh,
      scratch_types=[
          pltpu.VMEM((NB, K), jnp.int32),
          pltpu.VMEM((K, 128), f32),
          pltpu.VMEM_SHARED((NPAD, 128), f32),
      ],
  )
  def deg_kernel(dst_hbm, zeros_hbm, ones_hbm, out_hbm, dst_v, ones_v, acc):
    c = lax.axis_index("c")
    s = lax.axis_index("s")
    pltpu.sync_copy(dst_hbm.at[s], dst_v)
    pltpu.sync_copy(ones_hbm, ones_v)
    pltpu.sync_copy(zeros_hbm, acc.at[pl.ds(s * RPT, RPT)])
    plsc.subcore_barrier()

    half = NB // 2

    def body(j, carry):
      pltpu.sync_copy(ones_v, acc.at[dst_v.at[j]], add=True)
      return carry

    lax.fori_loop(c * half, half + c * (NB - half), body, 0)
    plsc.subcore_barrier()
    pltpu.sync_copy(acc.at[pl.ds(s * RPT, RPT)],
                    out_hbm.at[c, pl.ds(s * RPT, RPT)])

  return deg_kernel


def _scatter_kernel_fn(num_chunks):
  cpc = num_chunks // NC

  @functools.partial(
      pl.kernel,
      out_type=jax.ShapeDtypeStruct((NC, cpc, NPAD, 128), f32),
      mesh=_mesh,
      scratch_types=[
          pltpu.VMEM((EPT,), jnp.int32),
          pltpu.VMEM((NB, K), jnp.int32),
          pltpu.VMEM((K, 128), f32),
          pltpu.VMEM((K, 128), f32),
          pltpu.VMEM_SHARED((NPAD, 128), f32),
          pltpu.SemaphoreType.DMA,
          pltpu.SemaphoreType.DMA,
      ],
  )
  def scatter_kernel(m_hbm, src_hbm, dst_hbm, zeros_hbm, out_hbm,
                     src_v, dst_v, rows_a, rows_b, acc, sem_a, sem_b):
    c = lax.axis_index("c")
    s = lax.axis_index("s")
    pltpu.sync_copy(dst_hbm.at[s], dst_v)

    for cc_l in range(cpc):
      m_view = m_hbm
      pltpu.sync_copy(src_hbm.at[c * cpc + cc_l, s], src_v)
      pltpu.sync_copy(zeros_hbm, acc.at[pl.ds(s * RPT, RPT)])
      plsc.subcore_barrier()

      def start_g(j, buf, sem):
        pltpu.async_copy(m_view.at[src_v.at[pl.ds(j * K, K)]], buf, sem)

      def finish_g(j, buf, sem):
        pltpu.make_async_copy(m_view.at[src_v.at[pl.ds(j * K, K)]],
                              buf, sem).wait()

      def step(j, buf, sem):
        finish_g(j, buf, sem)
        pltpu.sync_copy(buf, acc.at[dst_v.at[j]], add=True)

      start_g(0, rows_a, sem_a)

      def body(j, carry):
        @pl.when(j % 2 == 0)
        def _():
          @pl.when(j + 1 < NB)
          def _():
            start_g(j + 1, rows_b, sem_b)
          step(j, rows_a, sem_a)

        @pl.when(j % 2 == 1)
        def _():
          @pl.when(j + 1 < NB)
          def _():
            start_g(j + 1, rows_a, sem_a)
          step(j, rows_b, sem_b)

        return carry

      lax.fori_loop(0, NB, body, 0)
      plsc.subcore_barrier()
      pltpu.sync_copy(acc.at[pl.ds(s * RPT, RPT)],
                      out_hbm.at[c, cc_l, pl.ds(s * RPT, RPT)])
      plsc.subcore_barrier()

  return scatter_kernel


_deg_call = _deg_kernel_fn()
_scatter4 = _scatter_kernel_fn(4)
_scatter2 = _scatter_kernel_fn(2)

RB = 1000


def _mm1_body(x_ref, w_ref, dinv_ref, o_ref):
  xs = x_ref[...] * dinv_ref[...]
  o_ref[0] = jnp.dot(xs, w_ref[...], preferred_element_type=f32)


def _mm2_body(hp_ref, m1_ref, dinv_ref, b1_ref, w2_ref, o_ref):
  k = pl.program_id(2)
  h = ((hp_ref[0] + m1_ref[0]) * dinv_ref[...]
       + b1_ref[...][None, :])
  part = jnp.dot(h, w2_ref[...], preferred_element_type=f32)

  @pl.when(k == 0)
  def _():
    o_ref[0] = part

  @pl.when(k > 0)
  def _():
    o_ref[0] += part

  @pl.when(k == (H // 128) - 1)
  def _():
    o_ref[0] *= dinv_ref[...]


def _fin_body(gp_ref, m2_ref, dinv_ref, b2_ref, o_ref):
  o_ref[...] = ((gp_ref[0] + m2_ref[0])
                * dinv_ref[...] + b2_ref[...][None, :])


def kernel(embedding, W1, b1, W2, b2, edge_index):
  dst = edge_index[1].reshape(NS, NB, K)

  zeros_acc = jnp.zeros((RPT, 128), f32)
  ones_deg = jnp.ones((K, 128), f32)

  degp = _deg_call(dst, zeros_acc, ones_deg)
  deg = 1.0 + degp[0, :, 0] + degp[1, :, 0]
  dinv = lax.rsqrt(deg)
  dinv_c = dinv.reshape(NPAD, 1)

  m1 = pl.pallas_call(
      _mm1_body,
      grid=(N // RB, H // 128),
      in_specs=[
          pl.BlockSpec((RB, D), lambda i, c: (i, 0)),
          pl.BlockSpec((D, 128), lambda i, c: (0, c)),
          pl.BlockSpec((RB, 1), lambda i, c: (i, 0)),
      ],
      out_specs=pl.BlockSpec((1, RB, 128), lambda i, c: (c, i, 0)),
      out_shape=jax.ShapeDtypeStruct((H // 128, N, 128), f32),
  )(embedding, W1, dinv_c)

  srcf = edge_index[0].reshape(NS, EPT)
  src4 = (srcf[None] + (jnp.arange(4, dtype=jnp.int32) * N)[:, None, None])
  acc1 = _scatter4(m1.reshape(4 * N, 128), src4, dst,
                   zeros_acc).reshape(4, NPAD, 128)

  m2 = pl.pallas_call(
      _mm2_body,
      grid=(N // RB, D // 128, H // 128),
      in_specs=[
          pl.BlockSpec((1, RB, 128), lambda i, c, k: (k, i, 0)),
          pl.BlockSpec((1, RB, 128), lambda i, c, k: (k, i, 0)),
          pl.BlockSpec((RB, 1), lambda i, c, k: (i, 0)),
          pl.BlockSpec((128,), lambda i, c, k: (k,)),
          pl.BlockSpec((128, 128), lambda i, c, k: (k, c)),
      ],
      out_specs=pl.BlockSpec((1, RB, 128), lambda i, c, k: (c, i, 0)),
      out_shape=jax.ShapeDtypeStruct((D // 128, N, 128), f32),
  )(acc1, m1, dinv_c, b1, W2)

  src2 = (srcf[None] + (jnp.arange(2, dtype=jnp.int32) * N)[:, None, None])
  acc2 = _scatter2(m2.reshape(2 * N, 128), src2, dst,
                   zeros_acc).reshape(2, NPAD, 128)

  out = pl.pallas_call(
      _fin_body,
      grid=(N // RB, D // 128),
      in_specs=[
          pl.BlockSpec((1, RB, 128), lambda i, c: (c, i, 0)),
          pl.BlockSpec((1, RB, 128), lambda i, c: (c, i, 0)),
          pl.BlockSpec((RB, 1), lambda i, c: (i, 0)),
          pl.BlockSpec((128,), lambda i, c: (c,)),
      ],
      out_specs=pl.BlockSpec((RB, 128), lambda i, c: (i, c)),
      out_shape=jax.ShapeDtypeStruct((N, D), f32),
  )(acc2, m2, dinv_c, b2)

  return out


# speedup vs baseline: 3.1458x; 1.0032x over previous
import functools

import jax
import jax.numpy as jnp
from jax import lax
from jax.experimental import pallas as pl
from jax.experimental.pallas import tpu as pltpu
from jax.experimental.pallas import tpu_sc as plsc

N = 10000
E = 160000
D = 256
H = 512

NC = 2
NS = 16
NW = NC * NS
K = 80
EPT = E // NS
NB = EPT // K
NPAD = 10240
RPT = NPAD // NS

_mesh = plsc.VectorSubcoreMesh(
    core_axis_name="c", subcore_axis_name="s", num_cores=NC, num_subcores=NS)

f32 = jnp.float32


def _deg_kernel_fn():
  @functools.partial(
      pl.kernel,
      out_type=jax.ShapeDtypeStruct((NC, NPAD, 128), f32),
      mesh=_mesh,
      scratch_types=[
          pltpu.VMEM((NB, K), jnp.int32),
          pltpu.VMEM((K, 128), f32),
          pltpu.VMEM_SHARED((NPAD, 128), f32),
      ],
  )
  def deg_kernel(dst_hbm, zeros_hbm, ones_hbm, out_hbm, dst_v, ones_v, acc):
    c = lax.axis_index("c")
    s = lax.axis_index("s")
    pltpu.sync_copy(dst_hbm.at[s], dst_v)
    pltpu.sync_copy(ones_hbm, ones_v)
    pltpu.sync_copy(zeros_hbm, acc.at[pl.ds(s * RPT, RPT)])
    plsc.subcore_barrier()

    half = NB // 2

    def body(j, carry):
      pltpu.sync_copy(ones_v, acc.at[dst_v.at[j]], add=True)
      return carry

    lax.fori_loop(c * half, half + c * (NB - half), body, 0)
    plsc.subcore_barrier()
    pltpu.sync_copy(acc.at[pl.ds(s * RPT, RPT)],
                    out_hbm.at[c, pl.ds(s * RPT, RPT)])

  return deg_kernel


def _scatter_kernel_fn(num_chunks):
  cpc = num_chunks // NC

  @functools.partial(
      pl.kernel,
      out_type=jax.ShapeDtypeStruct((NC, cpc, NPAD, 128), f32),
      mesh=_mesh,
      scratch_types=[
          pltpu.VMEM((EPT,), jnp.int32),
          pltpu.VMEM((NB, K), jnp.int32),
          pltpu.VMEM((K, 128), f32),
          pltpu.VMEM((K, 128), f32),
          pltpu.VMEM_SHARED((NPAD, 128), f32),
          pltpu.SemaphoreType.DMA,
          pltpu.SemaphoreType.DMA,
          pltpu.SemaphoreType.DMA,
          pltpu.SemaphoreType.DMA,
      ],
  )
  def scatter_kernel(m_hbm, src_hbm, dst_hbm, zeros_hbm, out_hbm,
                     src_v, dst_v, rows_a, rows_b, acc,
                     sem_ga, sem_gb, sem_sa, sem_sb):
    c = lax.axis_index("c")
    s = lax.axis_index("s")
    pltpu.sync_copy(dst_hbm.at[s], dst_v)

    for cc_l in range(cpc):
      m_view = m_hbm
      pltpu.sync_copy(src_hbm.at[c * cpc + cc_l, s], src_v)
      pltpu.sync_copy(zeros_hbm, acc.at[pl.ds(s * RPT, RPT)])
      plsc.subcore_barrier()

      def start_g(j, buf, sem):
        pltpu.async_copy(m_view.at[src_v.at[pl.ds(j * K, K)]], buf, sem)

      def finish_g(j, buf, sem):
        pltpu.make_async_copy(m_view.at[src_v.at[pl.ds(j * K, K)]],
                              buf, sem).wait()

      def start_s(j, buf, sem):
        pltpu.async_copy(buf, acc.at[dst_v.at[j]], sem, add=True)

      def finish_s(j, buf, sem):
        pltpu.make_async_copy(buf, acc.at[dst_v.at[j]], sem).wait()

      start_g(0, rows_a, sem_ga)

      def body(j, carry):
        @pl.when(j % 2 == 0)
        def _():
          @pl.when(j + 1 < NB)
          def _():
            @pl.when(j >= 1)
            def _():
              finish_s(j - 1, rows_b, sem_sb)
            start_g(j + 1, rows_b, sem_gb)
          finish_g(j, rows_a, sem_ga)
          start_s(j, rows_a, sem_sa)

        @pl.when(j % 2 == 1)
        def _():
          @pl.when(j + 1 < NB)
          def _():
            finish_s(j - 1, rows_a, sem_sa)
            start_g(j + 1, rows_a, sem_ga)
          finish_g(j, rows_b, sem_gb)
          start_s(j, rows_b, sem_sb)

        return carry

      lax.fori_loop(0, NB, body, 0)
      finish_s(NB - 2, rows_b, sem_sb)
      finish_s(NB - 1, rows_a, sem_sa)
      plsc.subcore_barrier()
      pltpu.sync_copy(acc.at[pl.ds(s * RPT, RPT)],
                      out_hbm.at[c, cc_l, pl.ds(s * RPT, RPT)])
      plsc.subcore_barrier()

  return scatter_kernel


_deg_call = _deg_kernel_fn()
_scatter4 = _scatter_kernel_fn(4)
_scatter2 = _scatter_kernel_fn(2)

RB = 1000


def _mm1_body(x_ref, w_ref, dinv_ref, o_ref):
  xs = x_ref[...] * dinv_ref[...]
  o_ref[0] = jnp.dot(xs, w_ref[...], preferred_element_type=f32)


def _mm2_body(hp_ref, m1_ref, dinv_ref, b1_ref, w2_ref, o_ref):
  k = pl.program_id(2)
  h = ((hp_ref[0] + m1_ref[0]) * dinv_ref[...]
       + b1_ref[...][None, :])
  part = jnp.dot(h, w2_ref[...], preferred_element_type=f32)

  @pl.when(k == 0)
  def _():
    o_ref[0] = part

  @pl.when(k > 0)
  def _():
    o_ref[0] += part

  @pl.when(k == (H // 128) - 1)
  def _():
    o_ref[0] *= dinv_ref[...]


def _fin_body(gp_ref, m2_ref, dinv_ref, b2_ref, o_ref):
  o_ref[...] = ((gp_ref[0] + m2_ref[0])
                * dinv_ref[...] + b2_ref[...][None, :])


def kernel(embedding, W1, b1, W2, b2, edge_index):
  dst = edge_index[1].reshape(NS, NB, K)

  zeros_acc = jnp.zeros((RPT, 128), f32)
  ones_deg = jnp.ones((K, 128), f32)

  degp = _deg_call(dst, zeros_acc, ones_deg)
  deg = 1.0 + degp[0, :, 0] + degp[1, :, 0]
  dinv = lax.rsqrt(deg)
  dinv_c = dinv.reshape(NPAD, 1)

  m1 = pl.pallas_call(
      _mm1_body,
      grid=(N // RB, H // 128),
      in_specs=[
          pl.BlockSpec((RB, D), lambda i, c: (i, 0)),
          pl.BlockSpec((D, 128), lambda i, c: (0, c)),
          pl.BlockSpec((RB, 1), lambda i, c: (i, 0)),
      ],
      out_specs=pl.BlockSpec((1, RB, 128), lambda i, c: (c, i, 0)),
      out_shape=jax.ShapeDtypeStruct((H // 128, N, 128), f32),
  )(embedding, W1, dinv_c)

  srcf = edge_index[0].reshape(NS, EPT)
  src4 = (srcf[None] + (jnp.arange(4, dtype=jnp.int32) * N)[:, None, None])
  acc1 = _scatter4(m1.reshape(4 * N, 128), src4, dst,
                   zeros_acc).reshape(4, NPAD, 128)

  m2 = pl.pallas_call(
      _mm2_body,
      grid=(N // RB, D // 128, H // 128),
      in_specs=[
          pl.BlockSpec((1, RB, 128), lambda i, c, k: (k, i, 0)),
          pl.BlockSpec((1, RB, 128), lambda i, c, k: (k, i, 0)),
          pl.BlockSpec((RB, 1), lambda i, c, k: (i, 0)),
          pl.BlockSpec((128,), lambda i, c, k: (k,)),
          pl.BlockSpec((128, 128), lambda i, c, k: (k, c)),
      ],
      out_specs=pl.BlockSpec((1, RB, 128), lambda i, c, k: (c, i, 0)),
      out_shape=jax.ShapeDtypeStruct((D // 128, N, 128), f32),
  )(acc1, m1, dinv_c, b1, W2)

  src2 = (srcf[None] + (jnp.arange(2, dtype=jnp.int32) * N)[:, None, None])
  acc2 = _scatter2(m2.reshape(2 * N, 128), src2, dst,
                   zeros_acc).reshape(2, NPAD, 128)

  out = pl.pallas_call(
      _fin_body,
      grid=(N // RB, D // 128),
      in_specs=[
          pl.BlockSpec((1, RB, 128), lambda i, c: (c, i, 0)),
          pl.BlockSpec((1, RB, 128), lambda i, c: (c, i, 0)),
          pl.BlockSpec((RB, 1), lambda i, c: (i, 0)),
          pl.BlockSpec((128,), lambda i, c: (c,)),
      ],
      out_specs=pl.BlockSpec((RB, 128), lambda i, c: (i, c)),
      out_shape=jax.ShapeDtypeStruct((N, D), f32),
  )(acc2, m2, dinv_c, b2)

  return out


# speedup vs baseline: 3.4675x; 1.1023x over previous
import functools

import jax
import jax.numpy as jnp
from jax import lax
from jax.experimental import pallas as pl
from jax.experimental.pallas import tpu as pltpu
from jax.experimental.pallas import tpu_sc as plsc

N = 10000
E = 160000
D = 256
H = 512

NC = 2
NS = 16
NW = NC * NS
K = 40
EPT = E // NS
NB = EPT // K
NBUF = 4
DEPTH = 3
NPAD = 10240
RPT = NPAD // NS

_mesh = plsc.VectorSubcoreMesh(
    core_axis_name="c", subcore_axis_name="s", num_cores=NC, num_subcores=NS)

f32 = jnp.float32


def _deg_kernel_fn():
  @functools.partial(
      pl.kernel,
      out_type=jax.ShapeDtypeStruct((NC, NPAD, 128), f32),
      mesh=_mesh,
      scratch_types=[
          pltpu.VMEM((EPT,), jnp.int32),
          pltpu.VMEM((K, 128), f32),
          pltpu.VMEM_SHARED((NPAD, 128), f32),
      ],
  )
  def deg_kernel(dst_hbm, zeros_hbm, ones_hbm, out_hbm, dst_v, ones_v, acc):
    c = lax.axis_index("c")
    s = lax.axis_index("s")
    pltpu.sync_copy(dst_hbm.at[s], dst_v)
    pltpu.sync_copy(ones_hbm, ones_v)
    pltpu.sync_copy(zeros_hbm, acc.at[pl.ds(s * RPT, RPT)])
    plsc.subcore_barrier()

    half = NB // 2

    def body(j, carry):
      pltpu.sync_copy(ones_v, acc.at[dst_v.at[pl.ds(j * K, K)]], add=True)
      return carry

    lax.fori_loop(c * half, half + c * (NB - half), body, 0)
    plsc.subcore_barrier()
    pltpu.sync_copy(acc.at[pl.ds(s * RPT, RPT)],
                    out_hbm.at[c, pl.ds(s * RPT, RPT)])

  return deg_kernel


def _scatter_kernel_fn(num_chunks):
  cpc = num_chunks // NC

  @functools.partial(
      pl.kernel,
      out_type=jax.ShapeDtypeStruct((NC, cpc, NPAD, 128), f32),
      mesh=_mesh,
      scratch_types=[
          pltpu.VMEM((EPT,), jnp.int32),
          pltpu.VMEM((EPT,), jnp.int32),
          pltpu.VMEM((K, 128), f32),
          pltpu.VMEM((K, 128), f32),
          pltpu.VMEM((K, 128), f32),
          pltpu.VMEM((K, 128), f32),
          pltpu.VMEM_SHARED((NPAD, 128), f32),
          [pltpu.SemaphoreType.DMA] * NBUF,
          [pltpu.SemaphoreType.DMA] * NBUF,
      ],
  )
  def scatter_kernel(m_hbm, src_hbm, dst_hbm, zeros_hbm, out_hbm,
                     src_v, dst_v, r0, r1, r2, r3, acc, sems_g, sems_s):
    c = lax.axis_index("c")
    s = lax.axis_index("s")
    rows = (r0, r1, r2, r3)
    pltpu.sync_copy(dst_hbm.at[s], dst_v)

    for cc_l in range(cpc):
      m_view = m_hbm
      pltpu.sync_copy(src_hbm.at[c * cpc + cc_l, s], src_v)
      pltpu.sync_copy(zeros_hbm, acc.at[pl.ds(s * RPT, RPT)])
      plsc.subcore_barrier()

      def start_g(j, b):
        pltpu.async_copy(m_view.at[src_v.at[pl.ds(j * K, K)]],
                         rows[b], sems_g[b])

      def finish_g(j, b):
        pltpu.make_async_copy(m_view.at[src_v.at[pl.ds(j * K, K)]],
                              rows[b], sems_g[b]).wait()

      def start_s(j, b):
        pltpu.async_copy(rows[b], acc.at[dst_v.at[pl.ds(j * K, K)]],
                         sems_s[b], add=True)

      def finish_s(j, b):
        pltpu.make_async_copy(rows[b], acc.at[dst_v.at[pl.ds(j * K, K)]],
                              sems_s[b]).wait()

      for t in range(DEPTH):
        start_g(t, t)

      def body(j, carry):
        for b in range(NBUF):
          @pl.when(j % NBUF == b)
          def _(b=b):
            bp = (b + DEPTH) % NBUF
            @pl.when(j + DEPTH < NB)
            def _():
              @pl.when(j >= 1)
              def _():
                finish_s(j - 1, bp)
              start_g(j + DEPTH, bp)
            finish_g(j, b)
            start_s(j, b)

        return carry

      lax.fori_loop(0, NB, body, 0)
      for t in range(NBUF):
        jj = NB - NBUF + t
        finish_s(jj, jj % NBUF)
      plsc.subcore_barrier()
      pltpu.sync_copy(acc.at[pl.ds(s * RPT, RPT)],
                      out_hbm.at[c, cc_l, pl.ds(s * RPT, RPT)])
      plsc.subcore_barrier()

  return scatter_kernel


_deg_call = _deg_kernel_fn()
_scatter4 = _scatter_kernel_fn(4)
_scatter2 = _scatter_kernel_fn(2)

RB = 1000


def _mm1_body(x_ref, w_ref, dinv_ref, o_ref):
  xs = x_ref[...] * dinv_ref[...]
  o_ref[0] = jnp.dot(xs, w_ref[...], preferred_element_type=f32)


def _mm2_body(hp_ref, m1_ref, dinv_ref, b1_ref, w2_ref, o_ref):
  k = pl.program_id(2)
  h = ((hp_ref[0] + m1_ref[0]) * dinv_ref[...]
       + b1_ref[...][None, :])
  part = jnp.dot(h, w2_ref[...], preferred_element_type=f32)

  @pl.when(k == 0)
  def _():
    o_ref[0] = part

  @pl.when(k > 0)
  def _():
    o_ref[0] += part

  @pl.when(k == (H // 128) - 1)
  def _():
    o_ref[0] *= dinv_ref[...]


def _fin_body(gp_ref, m2_ref, dinv_ref, b2_ref, o_ref):
  o_ref[...] = ((gp_ref[0] + m2_ref[0])
                * dinv_ref[...] + b2_ref[...][None, :])


def kernel(embedding, W1, b1, W2, b2, edge_index):
  dst = edge_index[1].reshape(NS, EPT)

  zeros_acc = jnp.zeros((RPT, 128), f32)
  ones_deg = jnp.ones((K, 128), f32)

  degp = _deg_call(dst, zeros_acc, ones_deg)
  deg = 1.0 + degp[0, :, 0] + degp[1, :, 0]
  dinv = lax.rsqrt(deg)
  dinv_c = dinv.reshape(NPAD, 1)

  m1 = pl.pallas_call(
      _mm1_body,
      grid=(N // RB, H // 128),
      in_specs=[
          pl.BlockSpec((RB, D), lambda i, c: (i, 0)),
          pl.BlockSpec((D, 128), lambda i, c: (0, c)),
          pl.BlockSpec((RB, 1), lambda i, c: (i, 0)),
      ],
      out_specs=pl.BlockSpec((1, RB, 128), lambda i, c: (c, i, 0)),
      out_shape=jax.ShapeDtypeStruct((H // 128, N, 128), f32),
  )(embedding, W1, dinv_c)

  srcf = edge_index[0].reshape(NS, EPT)
  src4 = (srcf[None] + (jnp.arange(4, dtype=jnp.int32) * N)[:, None, None])
  acc1 = _scatter4(m1.reshape(4 * N, 128), src4, dst,
                   zeros_acc).reshape(4, NPAD, 128)

  m2 = pl.pallas_call(
      _mm2_body,
      grid=(N // RB, D // 128, H // 128),
      in_specs=[
          pl.BlockSpec((1, RB, 128), lambda i, c, k: (k, i, 0)),
          pl.BlockSpec((1, RB, 128), lambda i, c, k: (k, i, 0)),
          pl.BlockSpec((RB, 1), lambda i, c, k: (i, 0)),
          pl.BlockSpec((128,), lambda i, c, k: (k,)),
          pl.BlockSpec((128, 128), lambda i, c, k: (k, c)),
      ],
      out_specs=pl.BlockSpec((1, RB, 128), lambda i, c, k: (c, i, 0)),
      out_shape=jax.ShapeDtypeStruct((D // 128, N, 128), f32),
  )(acc1, m1, dinv_c, b1, W2)

  src2 = (srcf[None] + (jnp.arange(2, dtype=jnp.int32) * N)[:, None, None])
  acc2 = _scatter2(m2.reshape(2 * N, 128), src2, dst,
                   zeros_acc).reshape(2, NPAD, 128)

  out = pl.pallas_call(
      _fin_body,
      grid=(N // RB, D // 128),
      in_specs=[
          pl.BlockSpec((1, RB, 128), lambda i, c: (c, i, 0)),
          pl.BlockSpec((1, RB, 128), lambda i, c: (c, i, 0)),
          pl.BlockSpec((RB, 1), lambda i, c: (i, 0)),
          pl.BlockSpec((128,), lambda i, c: (c,)),
      ],
      out_specs=pl.BlockSpec((RB, 128), lambda i, c: (i, c)),
      out_shape=jax.ShapeDtypeStruct((N, D), f32),
  )(acc2, m2, dinv_c, b2)

  return out


# speedup vs baseline: 3.5829x; 1.0333x over previous
import functools

import jax
import jax.numpy as jnp
from jax import lax
from jax.experimental import pallas as pl
from jax.experimental.pallas import tpu as pltpu
from jax.experimental.pallas import tpu_sc as plsc

N = 10000
E = 160000
D = 256
H = 512

NC = 2
NS = 16
NW = NC * NS
K = 40
EPT = E // NS
NB = EPT // K
NBUF = 5
DEPTH = 4
NPAD = 10240
RPT = NPAD // NS

_mesh = plsc.VectorSubcoreMesh(
    core_axis_name="c", subcore_axis_name="s", num_cores=NC, num_subcores=NS)

f32 = jnp.float32


def _deg_kernel_fn():
  @functools.partial(
      pl.kernel,
      out_type=jax.ShapeDtypeStruct((NC, NPAD, 128), f32),
      mesh=_mesh,
      scratch_types=[
          pltpu.VMEM((EPT,), jnp.int32),
          pltpu.VMEM((K, 128), f32),
          pltpu.VMEM_SHARED((NPAD, 128), f32),
      ],
  )
  def deg_kernel(dst_hbm, zeros_hbm, ones_hbm, out_hbm, dst_v, ones_v, acc):
    c = lax.axis_index("c")
    s = lax.axis_index("s")
    pltpu.sync_copy(dst_hbm.at[s], dst_v)
    pltpu.sync_copy(ones_hbm, ones_v)
    pltpu.sync_copy(zeros_hbm, acc.at[pl.ds(s * RPT, RPT)])
    plsc.subcore_barrier()

    half = NB // 2

    def body(j, carry):
      pltpu.sync_copy(ones_v, acc.at[dst_v.at[pl.ds(j * K, K)]], add=True)
      return carry

    lax.fori_loop(c * half, half + c * (NB - half), body, 0)
    plsc.subcore_barrier()
    pltpu.sync_copy(acc.at[pl.ds(s * RPT, RPT)],
                    out_hbm.at[c, pl.ds(s * RPT, RPT)])

  return deg_kernel


def _scatter_kernel_fn(num_chunks):
  cpc = num_chunks // NC

  @functools.partial(
      pl.kernel,
      out_type=jax.ShapeDtypeStruct((NC, cpc, NPAD, 128), f32),
      mesh=_mesh,
      scratch_types=[
          pltpu.VMEM((EPT,), jnp.int32),
          pltpu.VMEM((EPT,), jnp.int32),
          pltpu.VMEM((K, 128), f32),
          pltpu.VMEM((K, 128), f32),
          pltpu.VMEM((K, 128), f32),
          pltpu.VMEM((K, 128), f32),
          pltpu.VMEM((K, 128), f32),
          pltpu.VMEM_SHARED((NPAD, 128), f32),
          [pltpu.SemaphoreType.DMA] * NBUF,
          [pltpu.SemaphoreType.DMA] * NBUF,
      ],
  )
  def scatter_kernel(m_hbm, src_hbm, dst_hbm, zeros_hbm, out_hbm,
                     src_v, dst_v, r0, r1, r2, r3, r4, acc, sems_g, sems_s):
    c = lax.axis_index("c")
    s = lax.axis_index("s")
    rows = (r0, r1, r2, r3, r4)
    pltpu.sync_copy(dst_hbm.at[s], dst_v)

    for cc_l in range(cpc):
      m_view = m_hbm
      pltpu.sync_copy(src_hbm.at[c * cpc + cc_l, s], src_v)
      pltpu.sync_copy(zeros_hbm, acc.at[pl.ds(s * RPT, RPT)])
      plsc.subcore_barrier()

      def start_g(j, b):
        pltpu.async_copy(m_view.at[src_v.at[pl.ds(j * K, K)]],
                         rows[b], sems_g[b])

      def finish_g(j, b):
        pltpu.make_async_copy(m_view.at[src_v.at[pl.ds(j * K, K)]],
                              rows[b], sems_g[b]).wait()

      def start_s(j, b):
        pltpu.async_copy(rows[b], acc.at[dst_v.at[pl.ds(j * K, K)]],
                         sems_s[b], add=True)

      def finish_s(j, b):
        pltpu.make_async_copy(rows[b], acc.at[dst_v.at[pl.ds(j * K, K)]],
                              sems_s[b]).wait()

      for t in range(DEPTH):
        start_g(t, t)

      def body(j, carry):
        for b in range(NBUF):
          @pl.when(j % NBUF == b)
          def _(b=b):
            bp = (b + DEPTH) % NBUF
            @pl.when(j + DEPTH < NB)
            def _():
              @pl.when(j >= 1)
              def _():
                finish_s(j - 1, bp)
              start_g(j + DEPTH, bp)
            finish_g(j, b)
            start_s(j, b)

        return carry

      lax.fori_loop(0, NB, body, 0)
      for t in range(NBUF):
        jj = NB - NBUF + t
        finish_s(jj, jj % NBUF)
      plsc.subcore_barrier()
      pltpu.sync_copy(acc.at[pl.ds(s * RPT, RPT)],
                      out_hbm.at[c, cc_l, pl.ds(s * RPT, RPT)])
      plsc.subcore_barrier()

  return scatter_kernel


_deg_call = _deg_kernel_fn()
_scatter4 = _scatter_kernel_fn(4)
_scatter2 = _scatter_kernel_fn(2)

RB = 1000


def _mm1_body(x_ref, w_ref, dinv_ref, o_ref):
  xs = x_ref[...] * dinv_ref[...]
  o_ref[0] = jnp.dot(xs, w_ref[...], preferred_element_type=f32)


def _mm2_body(hp_ref, m1_ref, dinv_ref, b1_ref, w2_ref, o_ref):
  k = pl.program_id(2)
  h = ((hp_ref[0] + m1_ref[0]) * dinv_ref[...]
       + b1_ref[...][None, :])
  part = jnp.dot(h, w2_ref[...], preferred_element_type=f32)

  @pl.when(k == 0)
  def _():
    o_ref[0] = part

  @pl.when(k > 0)
  def _():
    o_ref[0] += part

  @pl.when(k == (H // 128) - 1)
  def _():
    o_ref[0] *= dinv_ref[...]


def _fin_body(gp_ref, m2_ref, dinv_ref, b2_ref, o_ref):
  o_ref[...] = ((gp_ref[0] + m2_ref[0])
                * dinv_ref[...] + b2_ref[...][None, :])


def kernel(embedding, W1, b1, W2, b2, edge_index):
  dst = edge_index[1].reshape(NS, EPT)

  zeros_acc = jnp.zeros((RPT, 128), f32)
  ones_deg = jnp.ones((K, 128), f32)

  degp = _deg_call(dst, zeros_acc, ones_deg)
  deg = 1.0 + degp[0, :, 0] + degp[1, :, 0]
  dinv = lax.rsqrt(deg)
  dinv_c = dinv.reshape(NPAD, 1)

  m1 = pl.pallas_call(
      _mm1_body,
      grid=(N // RB, H // 128),
      in_specs=[
          pl.BlockSpec((RB, D), lambda i, c: (i, 0)),
          pl.BlockSpec((D, 128), lambda i, c: (0, c)),
          pl.BlockSpec((RB, 1), lambda i, c: (i, 0)),
      ],
      out_specs=pl.BlockSpec((1, RB, 128), lambda i, c: (c, i, 0)),
      out_shape=jax.ShapeDtypeStruct((H // 128, N, 128), f32),
  )(embedding, W1, dinv_c)

  srcf = edge_index[0].reshape(NS, EPT)
  src4 = (srcf[None] + (jnp.arange(4, dtype=jnp.int32) * N)[:, None, None])
  acc1 = _scatter4(m1.reshape(4 * N, 128), src4, dst,
                   zeros_acc).reshape(4, NPAD, 128)

  m2 = pl.pallas_call(
      _mm2_body,
      grid=(N // RB, D // 128, H // 128),
      in_specs=[
          pl.BlockSpec((1, RB, 128), lambda i, c, k: (k, i, 0)),
          pl.BlockSpec((1, RB, 128), lambda i, c, k: (k, i, 0)),
          pl.BlockSpec((RB, 1), lambda i, c, k: (i, 0)),
          pl.BlockSpec((128,), lambda i, c, k: (k,)),
          pl.BlockSpec((128, 128), lambda i, c, k: (k, c)),
      ],
      out_specs=pl.BlockSpec((1, RB, 128), lambda i, c, k: (c, i, 0)),
      out_shape=jax.ShapeDtypeStruct((D // 128, N, 128), f32),
  )(acc1, m1, dinv_c, b1, W2)

  src2 = (srcf[None] + (jnp.arange(2, dtype=jnp.int32) * N)[:, None, None])
  acc2 = _scatter2(m2.reshape(2 * N, 128), src2, dst,
                   zeros_acc).reshape(2, NPAD, 128)

  out = pl.pallas_call(
      _fin_body,
      grid=(N // RB, D // 128),
      in_specs=[
          pl.BlockSpec((1, RB, 128), lambda i, c: (c, i, 0)),
          pl.BlockSpec((1, RB, 128), lambda i, c: (c, i, 0)),
          pl.BlockSpec((RB, 1), lambda i, c: (i, 0)),
          pl.BlockSpec((128,), lambda i, c: (c,)),
      ],
      out_specs=pl.BlockSpec((RB, 128), lambda i, c: (i, c)),
      out_shape=jax.ShapeDtypeStruct((N, D), f32),
  )(acc2, m2, dinv_c, b2)

  return out


# speedup vs baseline: 3.5854x; 1.0007x over previous
import functools

import jax
import jax.numpy as jnp
from jax import lax
from jax.experimental import pallas as pl
from jax.experimental.pallas import tpu as pltpu
from jax.experimental.pallas import tpu_sc as plsc

N = 10000
E = 160000
D = 256
H = 512

NC = 2
NS = 16
NW = NC * NS
K = 40
EPT = E // NS
NB = EPT // K
NBUF = 5
DEPTH = 4
NPAD = 10240
RPT = NPAD // NS

_mesh = plsc.VectorSubcoreMesh(
    core_axis_name="c", subcore_axis_name="s", num_cores=NC, num_subcores=NS)

f32 = jnp.float32


def _deg_kernel_fn():
  @functools.partial(
      pl.kernel,
      out_type=jax.ShapeDtypeStruct((NC, NPAD, 128), f32),
      mesh=_mesh,
      scratch_types=[
          pltpu.VMEM((EPT,), jnp.int32),
          pltpu.VMEM((K, 128), f32),
          pltpu.VMEM_SHARED((NPAD, 128), f32),
      ],
  )
  def deg_kernel(dst_hbm, zeros_hbm, ones_hbm, out_hbm, dst_v, ones_v, acc):
    c = lax.axis_index("c")
    s = lax.axis_index("s")
    pltpu.sync_copy(dst_hbm.at[s], dst_v)
    pltpu.sync_copy(ones_hbm, ones_v)
    pltpu.sync_copy(zeros_hbm, acc.at[pl.ds(s * RPT, RPT)])
    plsc.subcore_barrier()

    half = NB // 2

    def body(j, carry):
      pltpu.sync_copy(ones_v, acc.at[dst_v.at[pl.ds(j * K, K)]], add=True)
      return carry

    lax.fori_loop(c * half, half + c * (NB - half), body, 0)
    plsc.subcore_barrier()
    pltpu.sync_copy(acc.at[pl.ds(s * RPT, RPT)],
                    out_hbm.at[c, pl.ds(s * RPT, RPT)])

  return deg_kernel


def _scatter_kernel_fn(num_chunks):
  cpc = num_chunks // NC

  @functools.partial(
      pl.kernel,
      out_type=jax.ShapeDtypeStruct((NC, cpc, NPAD, 128), f32),
      mesh=_mesh,
      scratch_types=[
          pltpu.VMEM((EPT,), jnp.int32),
          pltpu.VMEM((EPT,), jnp.int32),
          pltpu.VMEM((K, 128), f32),
          pltpu.VMEM((K, 128), f32),
          pltpu.VMEM((K, 128), f32),
          pltpu.VMEM((K, 128), f32),
          pltpu.VMEM((K, 128), f32),
          pltpu.VMEM_SHARED((NPAD, 128), f32),
          [pltpu.SemaphoreType.DMA] * NBUF,
          [pltpu.SemaphoreType.DMA] * NBUF,
      ],
  )
  def scatter_kernel(m_hbm, src_hbm, dst_hbm, zeros_hbm, out_hbm,
                     src_v, dst_v, r0, r1, r2, r3, r4, acc, sems_g, sems_s):
    c = lax.axis_index("c")
    s = lax.axis_index("s")
    rows = (r0, r1, r2, r3, r4)
    pltpu.sync_copy(dst_hbm.at[s], dst_v)

    for cc_l in range(cpc):
      m_view = m_hbm
      pltpu.sync_copy(src_hbm.at[c * cpc + cc_l, s], src_v)
      pltpu.sync_copy(zeros_hbm, acc.at[pl.ds(s * RPT, RPT)])
      plsc.subcore_barrier()

      def start_g(j, b):
        pltpu.async_copy(m_view.at[src_v.at[pl.ds(j * K, K)]],
                         rows[b], sems_g[b])

      def finish_g(j, b):
        pltpu.make_async_copy(m_view.at[src_v.at[pl.ds(j * K, K)]],
                              rows[b], sems_g[b]).wait()

      def start_s(j, b):
        pltpu.async_copy(rows[b], acc.at[dst_v.at[pl.ds(j * K, K)]],
                         sems_s[b], add=True)

      def finish_s(j, b):
        pltpu.make_async_copy(rows[b], acc.at[dst_v.at[pl.ds(j * K, K)]],
                              sems_s[b]).wait()

      for t in range(DEPTH):
        start_g(t, t)

      def body(j, carry):
        for b in range(NBUF):
          @pl.when(j % NBUF == b)
          def _(b=b):
            bp = (b + DEPTH) % NBUF
            @pl.when(j + DEPTH < NB)
            def _():
              @pl.when(j >= 1)
              def _():
                finish_s(j - 1, bp)
              start_g(j + DEPTH, bp)
            finish_g(j, b)
            start_s(j, b)

        return carry

      lax.fori_loop(0, NB, body, 0)
      for t in range(NBUF):
        jj = NB - NBUF + t
        finish_s(jj, jj % NBUF)
      plsc.subcore_barrier()
      pltpu.sync_copy(acc.at[pl.ds(s * RPT, RPT)],
                      out_hbm.at[c, cc_l, pl.ds(s * RPT, RPT)])
      plsc.subcore_barrier()

  return scatter_kernel


_deg_call = _deg_kernel_fn()
_scatter4 = _scatter_kernel_fn(4)
_scatter2 = _scatter_kernel_fn(2)

RB = 1000


def _mm1_body(x_ref, w_ref, o_ref):
  o_ref[0] = jnp.dot(x_ref[...], w_ref[...], preferred_element_type=f32)


def _scale_body(m_ref, dinv_ref, o_ref):
  o_ref[0] = m_ref[0] * dinv_ref[...]


def _mm2_body(hp_ref, m1_ref, dinv_ref, b1_ref, w2_ref, o_ref):
  k = pl.program_id(2)
  h = ((hp_ref[0] + m1_ref[0]) * dinv_ref[...]
       + b1_ref[...][None, :])
  part = jnp.dot(h, w2_ref[...], preferred_element_type=f32)

  @pl.when(k == 0)
  def _():
    o_ref[0] = part

  @pl.when(k > 0)
  def _():
    o_ref[0] += part

  @pl.when(k == (H // 128) - 1)
  def _():
    o_ref[0] *= dinv_ref[...]


def _fin_body(gp_ref, m2_ref, dinv_ref, b2_ref, o_ref):
  o_ref[...] = ((gp_ref[0] + m2_ref[0])
                * dinv_ref[...] + b2_ref[...][None, :])


def kernel(embedding, W1, b1, W2, b2, edge_index):
  dst = edge_index[1].reshape(NS, EPT)

  zeros_acc = jnp.zeros((RPT, 128), f32)
  ones_deg = jnp.ones((K, 128), f32)

  degp = _deg_call(dst, zeros_acc, ones_deg)
  deg = 1.0 + degp[0, :, 0] + degp[1, :, 0]
  dinv = lax.rsqrt(deg)
  dinv_c = dinv.reshape(NPAD, 1)

  m1r = pl.pallas_call(
      _mm1_body,
      grid=(N // RB, H // 128),
      in_specs=[
          pl.BlockSpec((RB, D), lambda i, c: (i, 0)),
          pl.BlockSpec((D, 128), lambda i, c: (0, c)),
      ],
      out_specs=pl.BlockSpec((1, RB, 128), lambda i, c: (c, i, 0)),
      out_shape=jax.ShapeDtypeStruct((H // 128, N, 128), f32),
  )(embedding, W1)

  m1 = pl.pallas_call(
      _scale_body,
      grid=(N // RB, H // 128),
      in_specs=[
          pl.BlockSpec((1, RB, 128), lambda i, c: (c, i, 0)),
          pl.BlockSpec((RB, 1), lambda i, c: (i, 0)),
      ],
      out_specs=pl.BlockSpec((1, RB, 128), lambda i, c: (c, i, 0)),
      out_shape=jax.ShapeDtypeStruct((H // 128, N, 128), f32),
  )(m1r, dinv_c)

  srcf = edge_index[0].reshape(NS, EPT)
  src4 = (srcf[None] + (jnp.arange(4, dtype=jnp.int32) * N)[:, None, None])
  acc1 = _scatter4(m1.reshape(4 * N, 128), src4, dst,
                   zeros_acc).reshape(4, NPAD, 128)

  m2 = pl.pallas_call(
      _mm2_body,
      grid=(N // RB, D // 128, H // 128),
      in_specs=[
          pl.BlockSpec((1, RB, 128), lambda i, c, k: (k, i, 0)),
          pl.BlockSpec((1, RB, 128), lambda i, c, k: (k, i, 0)),
          pl.BlockSpec((RB, 1), lambda i, c, k: (i, 0)),
          pl.BlockSpec((128,), lambda i, c, k: (k,)),
          pl.BlockSpec((128, 128), lambda i, c, k: (k, c)),
      ],
      out_specs=pl.BlockSpec((1, RB, 128), lambda i, c, k: (c, i, 0)),
      out_shape=jax.ShapeDtypeStruct((D // 128, N, 128), f32),
  )(acc1, m1, dinv_c, b1, W2)

  src2 = (srcf[None] + (jnp.arange(2, dtype=jnp.int32) * N)[:, None, None])
  acc2 = _scatter2(m2.reshape(2 * N, 128), src2, dst,
                   zeros_acc).reshape(2, NPAD, 128)

  out = pl.pallas_call(
      _fin_body,
      grid=(N // RB, D // 128),
      in_specs=[
          pl.BlockSpec((1, RB, 128), lambda i, c: (c, i, 0)),
          pl.BlockSpec((1, RB, 128), lambda i, c: (c, i, 0)),
          pl.BlockSpec((RB, 1), lambda i, c: (i, 0)),
          pl.BlockSpec((128,), lambda i, c: (c,)),
      ],
      out_specs=pl.BlockSpec((RB, 128), lambda i, c: (i, c)),
      out_shape=jax.ShapeDtypeStruct((N, D), f32),
  )(acc2, m2, dinv_c, b2)

  return out
